# baseline (speedup 1.0000x reference)
import functools

import jax
import jax.numpy as jnp
from jax import lax
from jax.experimental import pallas as pl
from jax.experimental.pallas import tpu as pltpu
from jax.experimental.pallas import tpu_sc as plsc

N = 10000
E_NUM = 320000
D = 128

NC = 2
NS = 16
NW = NC * NS
E_PER_W = E_NUM // NW
CHUNK = 80
NCHUNK = E_PER_W // CHUNK
RPT = N // NS


def _deg_body(dst_hbm, ones_hbm, zeros_hbm, deg_out, idx_v, ones_v, acc, sem):
    del sem
    cid = lax.axis_index("c")
    sid = lax.axis_index("s")

    @pl.when(sid == 0)
    def _():
        pltpu.sync_copy(zeros_hbm, acc)

    pltpu.sync_copy(ones_hbm, ones_v)
    pltpu.sync_copy(dst_hbm.at[cid, sid], idx_v)
    plsc.subcore_barrier()

    def body(j, carry):
        pltpu.sync_copy(ones_v, acc.at[idx_v.at[j]], add=True)
        return carry

    lax.fori_loop(0, E_PER_W // CHUNK, body, 0)
    plsc.subcore_barrier()

    @pl.when(sid == 0)
    def _():
        pltpu.sync_copy(acc, deg_out.at[cid])


def _agg_body(src_hbm, dst_hbm, u3_hbm, u2_hbm, zeros_hbm, p_out,
              srcw, dst_v, buf0, buf1, p_acc, gsem):
    cid = lax.axis_index("c")
    sid = lax.axis_index("s")
    row0 = sid * RPT

    @pl.when(cid == 0)
    def _():
        pltpu.sync_copy(u3_hbm.at[sid], p_acc.at[pl.ds(row0, RPT)])

    @pl.when(cid != 0)
    def _():
        pltpu.sync_copy(zeros_hbm, p_acc.at[pl.ds(row0, RPT)])

    pltpu.sync_copy(dst_hbm.at[cid, sid], dst_v)

    def ldsrc(j, slot):
        pltpu.sync_copy(src_hbm.at[cid, sid, j], srcw.at[slot])

    def gather(slot, b):
        pltpu.make_async_copy(u2_hbm.at[srcw.at[slot]], b, gsem).start()

    def gwait(b):
        pltpu.make_async_copy(u2_hbm.at[srcw.at[0]], b, gsem).wait()

    def scat(j, b):
        pltpu.sync_copy(b, p_acc.at[dst_v.at[j]], add=True)

    ldsrc(0, 0)
    ldsrc(1, 1)
    plsc.subcore_barrier()

    gather(0, buf0)

    def pair(k, carry):
        j = 2 * k
        gwait(buf0)
        gather(1, buf1)
        scat(j, buf0)
        ldsrc(j + 2, 0)
        gwait(buf1)
        gather(0, buf0)
        scat(j + 1, buf1)

        @pl.when(k < (NCHUNK - 1) // 2 - 1)
        def _():
            ldsrc(j + 3, 1)

        return carry

    lax.fori_loop(0, (NCHUNK - 1) // 2, pair, 0)
    gwait(buf0)
    scat(NCHUNK - 1, buf0)
    plsc.subcore_barrier()
    pltpu.sync_copy(p_acc.at[pl.ds(row0, RPT)], p_out.at[cid, sid])


def _k2a_body(dp_ref, s_ref):
    deg = dp_ref[0:1, :] + dp_ref[1:2, :] + 1.0
    s_ref[...] = lax.rsqrt(jnp.maximum(deg, 1e-12))


def _k0_body(x_ref, w_ref, y_ref):
    y_ref[0] = lax.dot_general(x_ref[0], w_ref[...], (((1,), (0,)), ((), ())),
                               preferred_element_type=jnp.float32)


def _k2b_body(x_ref, s_ref, u_ref):
    u_ref[0] = x_ref[0] * s_ref[0]


def _k4_body(p0_ref, p1_ref, s_ref, b_ref, o_ref):
    o_ref[0] = (p0_ref[0, 0] + p1_ref[0, 0]) * s_ref[0] + b_ref[...]


def kernel(V, E, X, W, b):
    del V
    mesh = plsc.VectorSubcoreMesh(core_axis_name="c", subcore_axis_name="s")

    src3 = E[0].reshape(NC, NS, NCHUNK, CHUNK)
    dst3 = E[1].reshape(NC, NS, NCHUNK, CHUNK)
    ones_chunk = jnp.ones((CHUNK,), jnp.float32)
    zeros_n = jnp.zeros((N,), jnp.float32)
    zeros_rows = jnp.zeros((RPT, D), jnp.float32)

    k1 = functools.partial(
        pl.kernel,
        mesh=mesh,
        out_type=jax.ShapeDtypeStruct((NC, N), jnp.float32),
        scratch_types=[
            pltpu.VMEM((NCHUNK, CHUNK), jnp.int32),
            pltpu.VMEM((CHUNK,), jnp.float32),
            pltpu.VMEM_SHARED((N,), jnp.float32),
            pltpu.SemaphoreType.DMA,
        ],
    )(_deg_body)
    degp = k1(dst3, ones_chunk, zeros_n)

    s_row = pl.pallas_call(
        _k2a_body,
        out_shape=jax.ShapeDtypeStruct((1, N), jnp.float32),
    )(degp)

    X3 = X.reshape(NS, RPT, D)
    Y3 = pl.pallas_call(
        _k0_body,
        grid=(NS,),
        in_specs=[
            pl.BlockSpec((1, RPT, D), lambda i: (i, 0, 0)),
            pl.BlockSpec((D, D), lambda i: (0, 0)),
        ],
        out_specs=pl.BlockSpec((1, RPT, D), lambda i: (i, 0, 0)),
        out_shape=jax.ShapeDtypeStruct((NS, RPT, D), jnp.float32),
    )(X3, W)

    s3 = s_row.reshape(NS, RPT, 1)
    U3 = pl.pallas_call(
        _k2b_body,
        grid=(NS,),
        in_specs=[
            pl.BlockSpec((1, RPT, D), lambda i: (i, 0, 0)),
            pl.BlockSpec((1, RPT, 1), lambda i: (i, 0, 0)),
        ],
        out_specs=pl.BlockSpec((1, RPT, D), lambda i: (i, 0, 0)),
        out_shape=jax.ShapeDtypeStruct((NS, RPT, D), jnp.float32),
    )(Y3, s3)
    U2 = U3.reshape(N, D)

    k3 = functools.partial(
        pl.kernel,
        mesh=mesh,
        out_type=jax.ShapeDtypeStruct((NC, NS, RPT, D), jnp.float32),
        scratch_types=[
            pltpu.VMEM((2, CHUNK), jnp.int32),
            pltpu.VMEM((NCHUNK, CHUNK), jnp.int32),
            pltpu.VMEM((CHUNK, D), jnp.float32),
            pltpu.VMEM((CHUNK, D), jnp.float32),
            pltpu.VMEM_SHARED((N, D), jnp.float32),
            pltpu.SemaphoreType.DMA,
        ],
    )(_agg_body)
    Pp = k3(src3, dst3, U3, U2, zeros_rows)

    out = pl.pallas_call(
        _k4_body,
        grid=(NS,),
        in_specs=[
            pl.BlockSpec((1, 1, RPT, D), lambda i: (0, i, 0, 0)),
            pl.BlockSpec((1, 1, RPT, D), lambda i: (1, i, 0, 0)),
            pl.BlockSpec((1, RPT, 1), lambda i: (i, 0, 0)),
            pl.BlockSpec((1, D), lambda i: (0, 0)),
        ],
        out_specs=pl.BlockSpec((1, RPT, D), lambda i: (i, 0, 0)),
        out_shape=jax.ShapeDtypeStruct((NS, RPT, D), jnp.float32),
    )(Pp, Pp, s3, b.reshape(1, D))
    return out.reshape(N, D)


# speedup vs baseline: 1.0105x; 1.0105x over previous
import functools

import jax
import jax.numpy as jnp
from jax import lax
from jax.experimental import pallas as pl
from jax.experimental.pallas import tpu as pltpu
from jax.experimental.pallas import tpu_sc as plsc

N = 10000
E_NUM = 320000
D = 128

NC = 2
NS = 16
NW = NC * NS
E_PER_W = E_NUM // NW
CHUNK = 80
NCHUNK = E_PER_W // CHUNK
RPT = N // NS


def _deg_body(dst_hbm, ones_hbm, zeros_hbm, deg_out, idx_v, ones_v, acc, sem):
    del sem
    cid = lax.axis_index("c")
    sid = lax.axis_index("s")

    @pl.when(sid == 0)
    def _():
        pltpu.sync_copy(zeros_hbm, acc)

    pltpu.sync_copy(ones_hbm, ones_v)
    pltpu.sync_copy(dst_hbm.at[cid, sid], idx_v)
    plsc.subcore_barrier()

    def body(j, carry):
        pltpu.sync_copy(ones_v, acc.at[idx_v.at[j]], add=True)
        return carry

    lax.fori_loop(0, E_PER_W // CHUNK, body, 0)
    plsc.subcore_barrier()

    @pl.when(sid == 0)
    def _():
        pltpu.sync_copy(acc, deg_out.at[cid])


def _agg_body(src_hbm, dst_hbm, u3_hbm, u2_hbm, zeros_hbm, p_out,
              srcw, dst_v, buf0, buf1, p_acc, gsem, isem):
    cid = lax.axis_index("c")
    sid = lax.axis_index("s")
    row0 = sid * RPT

    @pl.when(cid == 0)
    def _():
        pltpu.sync_copy(u3_hbm.at[sid], p_acc.at[pl.ds(row0, RPT)])

    @pl.when(cid != 0)
    def _():
        pltpu.sync_copy(zeros_hbm, p_acc.at[pl.ds(row0, RPT)])

    pltpu.sync_copy(dst_hbm.at[cid, sid], dst_v)

    def ldsrc(j, slot):
        pltpu.make_async_copy(src_hbm.at[cid, sid, j], srcw.at[slot], isem).start()

    def iwait():
        pltpu.make_async_copy(src_hbm.at[cid, sid, 0], srcw.at[0], isem).wait()

    def gather(slot, b):
        pltpu.make_async_copy(u2_hbm.at[srcw.at[slot]], b, gsem).start()

    def gwait(b):
        pltpu.make_async_copy(u2_hbm.at[srcw.at[0]], b, gsem).wait()

    def scat(j, b):
        pltpu.sync_copy(b, p_acc.at[dst_v.at[j]], add=True)

    ldsrc(0, 0)
    ldsrc(1, 1)
    iwait()
    iwait()
    plsc.subcore_barrier()

    gather(0, buf0)

    def pair(k, carry):
        j = 2 * k
        gwait(buf0)
        gather(1, buf1)
        ldsrc(j + 2, 0)
        scat(j, buf0)
        iwait()
        gwait(buf1)
        gather(0, buf0)

        @pl.when(k < (NCHUNK - 1) // 2 - 1)
        def _():
            ldsrc(j + 3, 1)

        scat(j + 1, buf1)

        @pl.when(k < (NCHUNK - 1) // 2 - 1)
        def _():
            iwait()

        return carry

    lax.fori_loop(0, (NCHUNK - 1) // 2, pair, 0)
    gwait(buf0)
    scat(NCHUNK - 1, buf0)
    plsc.subcore_barrier()
    pltpu.sync_copy(p_acc.at[pl.ds(row0, RPT)], p_out.at[cid, sid])


def _k2a_body(dp_ref, s_ref):
    deg = dp_ref[0:1, :] + dp_ref[1:2, :] + 1.0
    s_ref[...] = lax.rsqrt(jnp.maximum(deg, 1e-12))


def _k2b_body(x_ref, s_ref, u_ref):
    u_ref[0] = x_ref[0] * s_ref[0]


def _k4_body(p0_ref, p1_ref, s_ref, w_ref, b_ref, o_ref):
    agg = (p0_ref[0, 0] + p1_ref[0, 0]) * s_ref[0]
    o_ref[0] = (
        lax.dot_general(agg, w_ref[...], (((1,), (0,)), ((), ())),
                        preferred_element_type=jnp.float32)
        + b_ref[...]
    )


def kernel(V, E, X, W, b):
    del V
    mesh = plsc.VectorSubcoreMesh(core_axis_name="c", subcore_axis_name="s")

    src3 = E[0].reshape(NC, NS, NCHUNK, CHUNK)
    dst3 = E[1].reshape(NC, NS, NCHUNK, CHUNK)
    ones_chunk = jnp.ones((CHUNK,), jnp.float32)
    zeros_n = jnp.zeros((N,), jnp.float32)
    zeros_rows = jnp.zeros((RPT, D), jnp.float32)

    k1 = functools.partial(
        pl.kernel,
        mesh=mesh,
        out_type=jax.ShapeDtypeStruct((NC, N), jnp.float32),
        scratch_types=[
            pltpu.VMEM((NCHUNK, CHUNK), jnp.int32),
            pltpu.VMEM((CHUNK,), jnp.float32),
            pltpu.VMEM_SHARED((N,), jnp.float32),
            pltpu.SemaphoreType.DMA,
        ],
    )(_deg_body)
    degp = k1(dst3, ones_chunk, zeros_n)

    s_row = pl.pallas_call(
        _k2a_body,
        out_shape=jax.ShapeDtypeStruct((1, N), jnp.float32),
    )(degp)

    s3 = s_row.reshape(NS, RPT, 1)
    X3 = X.reshape(NS, RPT, D)
    U3 = pl.pallas_call(
        _k2b_body,
        grid=(NS,),
        in_specs=[
            pl.BlockSpec((1, RPT, D), lambda i: (i, 0, 0)),
            pl.BlockSpec((1, RPT, 1), lambda i: (i, 0, 0)),
        ],
        out_specs=pl.BlockSpec((1, RPT, D), lambda i: (i, 0, 0)),
        out_shape=jax.ShapeDtypeStruct((NS, RPT, D), jnp.float32),
    )(X3, s3)
    U2 = U3.reshape(N, D)

    k3 = functools.partial(
        pl.kernel,
        mesh=mesh,
        out_type=jax.ShapeDtypeStruct((NC, NS, RPT, D), jnp.float32),
        scratch_types=[
            pltpu.VMEM((2, CHUNK), jnp.int32),
            pltpu.VMEM((NCHUNK, CHUNK), jnp.int32),
            pltpu.VMEM((CHUNK, D), jnp.float32),
            pltpu.VMEM((CHUNK, D), jnp.float32),
            pltpu.VMEM_SHARED((N, D), jnp.float32),
            pltpu.SemaphoreType.DMA,
            pltpu.SemaphoreType.DMA,
        ],
    )(_agg_body)
    Pp = k3(src3, dst3, U3, U2, zeros_rows)

    out = pl.pallas_call(
        _k4_body,
        grid=(NS,),
        in_specs=[
            pl.BlockSpec((1, 1, RPT, D), lambda i: (0, i, 0, 0)),
            pl.BlockSpec((1, 1, RPT, D), lambda i: (1, i, 0, 0)),
            pl.BlockSpec((1, RPT, 1), lambda i: (i, 0, 0)),
            pl.BlockSpec((D, D), lambda i: (0, 0)),
            pl.BlockSpec((1, D), lambda i: (0, 0)),
        ],
        out_specs=pl.BlockSpec((1, RPT, D), lambda i: (i, 0, 0)),
        out_shape=jax.ShapeDtypeStruct((NS, RPT, D), jnp.float32),
    )(Pp, Pp, s3, W, b.reshape(1, D))
    return out.reshape(N, D)
